# Initial kernel scaffold; baseline (speedup 1.0000x reference)
import jax
import jax.numpy as jnp
from jax.experimental import pallas as pl


def kernel(X, theta_w, theta_b, ln_w, ln_b):
    raise NotImplementedError("write your pallas kernel here")


# speedup vs baseline: 45.2331x; 45.2331x over previous
import functools

import jax
import jax.numpy as jnp
from jax.experimental import pallas as pl

_K = 8
_EPS = 1e-5
_ROWS = 256


def _body(x_ref, twt_ref, tb_ref, lw_ref, lb_ref, o_ref):
    i = pl.program_id(1)
    n, d = x_ref.shape[1], x_ref.shape[2]
    rows = o_ref.shape[1]

    x_full = x_ref[0]
    x_blk = x_ref[0, pl.ds(i * rows, rows), :]

    nrm_full = jnp.sqrt(jnp.sum(x_full * x_full, axis=1, keepdims=True))
    xn_full = x_full / jnp.maximum(nrm_full, 1e-12)
    nrm_blk = jnp.sqrt(jnp.sum(x_blk * x_blk, axis=1, keepdims=True))
    xn_blk = x_blk / jnp.maximum(nrm_blk, 1e-12)

    sim = jax.lax.dot_general(
        xn_blk.astype(jnp.bfloat16), xn_full.astype(jnp.bfloat16),
        (((1,), (1,)), ((), ())),
        preferred_element_type=jnp.float32)

    m = sim
    for _ in range(_K):
        v = jnp.max(m, axis=1, keepdims=True)
        m = jnp.where(m >= v, -jnp.inf, m)

    mask = (sim >= v).astype(jnp.bfloat16)
    msg = jax.lax.dot_general(
        mask, x_full.astype(jnp.bfloat16), (((1,), (0,)), ((), ())),
        preferred_element_type=jnp.float32) * (1.0 / _K)

    y = jax.lax.dot_general(
        msg.astype(jnp.bfloat16), twt_ref[...].astype(jnp.bfloat16),
        (((1,), (0,)), ((), ())),
        preferred_element_type=jnp.float32) + tb_ref[...]

    z = x_blk + y
    mu = jnp.mean(z, axis=1, keepdims=True)
    var = jnp.mean((z - mu) ** 2, axis=1, keepdims=True)
    o_ref[0] = (z - mu) / jnp.sqrt(var + _EPS) * lw_ref[...] + lb_ref[...]


@functools.partial(jax.jit, static_argnames=())
def kernel(X, theta_w, theta_b, ln_w, ln_b):
    B, N, D = X.shape
    rows = _ROWS
    grid = (B, N // rows)
    out = pl.pallas_call(
        _body,
        grid=grid,
        in_specs=[
            pl.BlockSpec((1, N, D), lambda b, i: (b, 0, 0)),
            pl.BlockSpec((D, D), lambda b, i: (0, 0)),
            pl.BlockSpec((1, D), lambda b, i: (0, 0)),
            pl.BlockSpec((1, D), lambda b, i: (0, 0)),
            pl.BlockSpec((1, D), lambda b, i: (0, 0)),
        ],
        out_specs=pl.BlockSpec((1, rows, D), lambda b, i: (b, i, 0)),
        out_shape=jax.ShapeDtypeStruct((B, N, D), jnp.float32),
    )(X, theta_w.T, theta_b.reshape(1, D), ln_w.reshape(1, D),
      ln_b.reshape(1, D))
    return out


# speedup vs baseline: 73.2264x; 1.6189x over previous
import functools

import jax
import jax.numpy as jnp
from jax.experimental import pallas as pl
from jax.experimental.pallas import tpu as pltpu

_K = 8
_EPS = 1e-5
_ROWS = 256
_LANES = 128
_NEG = float("-inf")


def _kth_largest(m, k):
    v = None
    for _ in range(k):
        v = jnp.max(m, axis=1, keepdims=True)
        m = jnp.where(m >= v, _NEG, m)
    return v


def _body(x_ref, twt_ref, tb_ref, lw_ref, lb_ref, o_ref, xn_ref, xb_ref,
          t_ref):
    i = pl.program_id(1)
    n = x_ref.shape[1]
    rows = o_ref.shape[1]

    @pl.when(i == 0)
    def _():
        xf = x_ref[0]
        nrm = jnp.sqrt(jnp.sum(xf * xf, axis=1, keepdims=True))
        xn_ref[...] = (xf / jnp.maximum(nrm, 1e-12)).astype(jnp.bfloat16)
        xb_ref[...] = xf.astype(jnp.bfloat16)

    x_blk = x_ref[0, pl.ds(i * rows, rows), :]
    xn_blk = xn_ref[pl.ds(i * rows, rows), :]

    sim = jax.lax.dot_general(
        xn_blk, xn_ref[...], (((1,), (1,)), ((), ())),
        preferred_element_type=jnp.float32)

    r1 = sim[:, 0:_LANES]
    r2 = jnp.full_like(r1, _NEG)
    r3 = jnp.full_like(r1, _NEG)
    for c in range(1, n // _LANES):
        x = sim[:, c * _LANES:(c + 1) * _LANES]
        b1 = jnp.minimum(r1, x)
        r1 = jnp.maximum(r1, x)
        b2 = jnp.minimum(r2, b1)
        r2 = jnp.maximum(r2, b1)
        r3 = jnp.maximum(r3, b2)
    cand = jnp.concatenate([r1, r2, r3], axis=1)
    v = _kth_largest(cand, _K)
    cnt = jnp.sum((sim >= v).astype(jnp.float32), axis=1, keepdims=True)
    t_ref[...] = v

    @pl.when(jnp.any(cnt != float(_K)))
    def _():
        t_ref[...] = _kth_largest(sim, _K)

    mask = (sim >= t_ref[...]).astype(jnp.bfloat16)
    msg = jax.lax.dot_general(
        mask, xb_ref[...], (((1,), (0,)), ((), ())),
        preferred_element_type=jnp.float32) * (1.0 / _K)

    y = jax.lax.dot_general(
        msg.astype(jnp.bfloat16), twt_ref[...].astype(jnp.bfloat16),
        (((1,), (0,)), ((), ())),
        preferred_element_type=jnp.float32) + tb_ref[...]

    z = x_blk + y
    mu = jnp.mean(z, axis=1, keepdims=True)
    var = jnp.mean((z - mu) ** 2, axis=1, keepdims=True)
    o_ref[0] = (z - mu) / jnp.sqrt(var + _EPS) * lw_ref[...] + lb_ref[...]


@functools.partial(jax.jit, static_argnames=())
def kernel(X, theta_w, theta_b, ln_w, ln_b):
    B, N, D = X.shape
    rows = _ROWS
    grid = (B, N // rows)
    out = pl.pallas_call(
        _body,
        grid=grid,
        in_specs=[
            pl.BlockSpec((1, N, D), lambda b, i: (b, 0, 0)),
            pl.BlockSpec((D, D), lambda b, i: (0, 0)),
            pl.BlockSpec((1, D), lambda b, i: (0, 0)),
            pl.BlockSpec((1, D), lambda b, i: (0, 0)),
            pl.BlockSpec((1, D), lambda b, i: (0, 0)),
        ],
        out_specs=pl.BlockSpec((1, rows, D), lambda b, i: (b, i, 0)),
        out_shape=jax.ShapeDtypeStruct((B, N, D), jnp.float32),
        scratch_shapes=[
            pltpu.VMEM((N, D), jnp.bfloat16),
            pltpu.VMEM((N, D), jnp.bfloat16),
            pltpu.VMEM((rows, 1), jnp.float32),
        ],
    )(X, theta_w.T, theta_b.reshape(1, D), ln_w.reshape(1, D),
      ln_b.reshape(1, D))
    return out


# speedup vs baseline: 79.9333x; 1.0916x over previous
import functools

import jax
import jax.numpy as jnp
from jax.experimental import pallas as pl
from jax.experimental.pallas import tpu as pltpu

_K = 8
_EPS = 1e-5
_ROWS = 256
_LANES = 128
_NEG = float("-inf")


def _kth_largest(m, k):
    v = None
    for _ in range(k):
        v = jnp.max(m, axis=1, keepdims=True)
        m = jnp.where(m >= v, _NEG, m)
    return v


def _body(x_ref, twt_ref, tb_ref, lw_ref, lb_ref, o_ref, xn_ref, xb_ref,
          msg_ref):
    i = pl.program_id(1)
    n = x_ref.shape[1]
    d = x_ref.shape[2]
    rows = o_ref.shape[1]

    @pl.when(i == 0)
    def _():
        xf = x_ref[0]
        nrm = jnp.sqrt(jnp.sum(xf * xf, axis=1, keepdims=True))
        xn_ref[...] = (xf / jnp.maximum(nrm, 1e-12)).astype(jnp.bfloat16)
        pad = (jax.lax.broadcasted_iota(jnp.int32, (n, d), 1) == 0)
        xb_ref[...] = jnp.concatenate(
            [xf.astype(jnp.bfloat16), pad.astype(jnp.bfloat16)], axis=1)

    x_blk = x_ref[0, pl.ds(i * rows, rows), :]
    xn_blk = xn_ref[pl.ds(i * rows, rows), :]

    sim = jax.lax.dot_general(
        xn_blk, xn_ref[...], (((1,), (1,)), ((), ())),
        preferred_element_type=jnp.float32)

    r1 = sim[:, 0:_LANES]
    r2 = jnp.full_like(r1, _NEG)
    r3 = jnp.full_like(r1, _NEG)
    for c in range(1, n // _LANES):
        x = sim[:, c * _LANES:(c + 1) * _LANES]
        b1 = jnp.minimum(r1, x)
        r1 = jnp.maximum(r1, x)
        b2 = jnp.minimum(r2, b1)
        r2 = jnp.maximum(r2, b1)
        r3 = jnp.maximum(r3, b2)
    cand = jnp.concatenate([r1, r2, r3], axis=1)
    v = _kth_largest(cand, _K)
    mask = (sim >= v).astype(jnp.bfloat16)
    msg_ref[...] = jax.lax.dot_general(
        mask, xb_ref[...], (((1,), (0,)), ((), ())),
        preferred_element_type=jnp.float32)
    cnt = msg_ref[:, d:d + 1]

    @pl.when(jnp.any(cnt != float(_K)))
    def _():
        t = _kth_largest(sim, _K)
        mask2 = (sim >= t).astype(jnp.bfloat16)
        msg_ref[...] = jax.lax.dot_general(
            mask2, xb_ref[...], (((1,), (0,)), ((), ())),
            preferred_element_type=jnp.float32)

    msg = msg_ref[:, :d] * (1.0 / _K)
    y = jax.lax.dot_general(
        msg.astype(jnp.bfloat16), twt_ref[...].astype(jnp.bfloat16),
        (((1,), (0,)), ((), ())),
        preferred_element_type=jnp.float32) + tb_ref[...]

    z = x_blk + y
    mu = jnp.mean(z, axis=1, keepdims=True)
    var = jnp.mean((z - mu) ** 2, axis=1, keepdims=True)
    o_ref[0] = (z - mu) / jnp.sqrt(var + _EPS) * lw_ref[...] + lb_ref[...]


@functools.partial(jax.jit, static_argnames=())
def kernel(X, theta_w, theta_b, ln_w, ln_b):
    B, N, D = X.shape
    rows = _ROWS
    grid = (B, N // rows)
    out = pl.pallas_call(
        _body,
        grid=grid,
        in_specs=[
            pl.BlockSpec((1, N, D), lambda b, i: (b, 0, 0)),
            pl.BlockSpec((D, D), lambda b, i: (0, 0)),
            pl.BlockSpec((1, D), lambda b, i: (0, 0)),
            pl.BlockSpec((1, D), lambda b, i: (0, 0)),
            pl.BlockSpec((1, D), lambda b, i: (0, 0)),
        ],
        out_specs=pl.BlockSpec((1, rows, D), lambda b, i: (b, i, 0)),
        out_shape=jax.ShapeDtypeStruct((B, N, D), jnp.float32),
        scratch_shapes=[
            pltpu.VMEM((N, D), jnp.bfloat16),
            pltpu.VMEM((N, 2 * D), jnp.bfloat16),
            pltpu.VMEM((rows, 2 * D), jnp.float32),
        ],
    )(X, theta_w.T, theta_b.reshape(1, D), ln_w.reshape(1, D),
      ln_b.reshape(1, D))
    return out
